# baseline (speedup 1.0000x reference)
import functools

import jax
import jax.numpy as jnp
from jax import lax
from jax.experimental import pallas as pl
from jax.experimental.pallas import tpu as pltpu, tpu_sc as plsc

_SCALE = 10 - 0.0001
_NEG_LOG2E = -1.4426950408889634
_N = 4194304
_NW = 32
_PER_W = _N // _NW
_C = 16384
_CHUNKS = _PER_W // _C
_L = 16
_UNROLL = 16


def _sc_body(x_hbm, t_hbm, o_hbm,
             xb0, xb1, tb0, tb1, ob0, ob1,
             sx0, sx1, st0, st1, so0, so1):
    wid = lax.axis_index("s") * 2 + lax.axis_index("c")
    base = wid * _PER_W
    xbufs, tbufs, obufs = (xb0, xb1), (tb0, tb1), (ob0, ob1)
    sxs, sts, sos = (sx0, sx1), (st0, st1), (so0, so1)

    def start_in(c, b):
        off = base + c * _C
        pltpu.async_copy(x_hbm.at[pl.ds(off, _C)], xbufs[b], sxs[b])
        pltpu.async_copy(t_hbm.at[pl.ds(off, _C)], tbufs[b], sts[b])

    start_in(0, 0)
    start_in(1, 1)

    @pl.loop(0, _CHUNKS // 2)
    def _chunks(g):
        for b in range(2):
            c = g * 2 + b
            xb, tb, ob = xbufs[b], tbufs[b], obufs[b]
            pltpu.make_async_copy(x_hbm.at[pl.ds(0, _C)], xb, sxs[b]).wait()
            pltpu.make_async_copy(t_hbm.at[pl.ds(0, _C)], tb, sts[b]).wait()
            @pl.when(g >= 1)
            def _():
                pltpu.make_async_copy(
                    ob, o_hbm.at[pl.ds(base, _C)], sos[b]).wait()

            @plsc.parallel_loop(0, _C, step=_L, unroll=_UNROLL)
            def _compute(s):
                xv = xb[pl.ds(s, _L)]
                tv = tb[pl.ds(s, _L)]
                sig = 1.0 / (1.0 + jnp.exp(xv * -1.0))
                g_ = jnp.abs(sig - tv)
                ob[pl.ds(s, _L)] = (g_ * _SCALE).astype(jnp.int32)

            pltpu.async_copy(ob, o_hbm.at[pl.ds(base + c * _C, _C)], sos[b])
            @pl.when(c + 2 < _CHUNKS)
            def _():
                start_in(c + 2, b)

    for b in range(2):
        pltpu.make_async_copy(obufs[b], o_hbm.at[pl.ds(base, _C)],
                              sos[b]).wait()


@jax.jit
def kernel(x, target):
    mesh = plsc.VectorSubcoreMesh(core_axis_name="c", subcore_axis_name="s")
    run = functools.partial(
        pl.kernel,
        mesh=mesh,
        compiler_params=pltpu.CompilerParams(
            use_tc_tiling_on_sc=True,
            skip_device_barrier=True,
            disable_bounds_checks=True,
            disable_semaphore_checks=True,
        ),
        out_type=jax.ShapeDtypeStruct((_N,), jnp.int32),
        scratch_types=[
            pltpu.VMEM((_C,), jnp.float32),
            pltpu.VMEM((_C,), jnp.float32),
            pltpu.VMEM((_C,), jnp.float32),
            pltpu.VMEM((_C,), jnp.float32),
            pltpu.VMEM((_C,), jnp.int32),
            pltpu.VMEM((_C,), jnp.int32),
            pltpu.SemaphoreType.DMA,
            pltpu.SemaphoreType.DMA,
            pltpu.SemaphoreType.DMA,
            pltpu.SemaphoreType.DMA,
            pltpu.SemaphoreType.DMA,
            pltpu.SemaphoreType.DMA,
        ],
    )(_sc_body)
    return run(x, target)


# speedup vs baseline: 1.0626x; 1.0626x over previous
import functools

import jax
import jax.numpy as jnp
from jax import lax
from jax.experimental import pallas as pl
from jax.experimental.pallas import tpu as pltpu, tpu_sc as plsc

_SCALE = 10 - 0.0001
_N = 4194304
_NW = 32
_PER_W = _N // _NW
_C = 8192
_CHUNKS = _PER_W // _C
_NB = 4
_L = 16
_UNROLL = 8


def _sc_body(x_hbm, t_hbm, o_hbm, xbufs, tbufs, obufs, sxs, sts, sos):
    wid = lax.axis_index("s") * 2 + lax.axis_index("c")
    base = wid * _PER_W

    def start_in(c, b):
        off = base + c * _C
        pltpu.async_copy(x_hbm.at[pl.ds(off, _C)], xbufs[b], sxs[b])
        pltpu.async_copy(t_hbm.at[pl.ds(off, _C)], tbufs[b], sts[b])

    for b in range(_NB):
        start_in(b, b)

    @pl.loop(0, _CHUNKS // _NB)
    def _chunks(g):
        for b in range(_NB):
            c = g * _NB + b
            xb, tb, ob = xbufs[b], tbufs[b], obufs[b]
            pltpu.make_async_copy(x_hbm.at[pl.ds(0, _C)], xb, sxs[b]).wait()
            pltpu.make_async_copy(t_hbm.at[pl.ds(0, _C)], tb, sts[b]).wait()
            @pl.when(g >= 1)
            def _():
                pltpu.make_async_copy(
                    ob, o_hbm.at[pl.ds(base, _C)], sos[b]).wait()

            @plsc.parallel_loop(0, _C, step=_L, unroll=_UNROLL)
            def _compute(s):
                xv = xb[pl.ds(s, _L)]
                tv = tb[pl.ds(s, _L)]
                sig = 1.0 / (1.0 + jnp.exp(xv * -1.0))
                g_ = jnp.abs(sig - tv)
                ob[pl.ds(s, _L)] = (g_ * _SCALE).astype(jnp.int32)

            pltpu.async_copy(ob, o_hbm.at[pl.ds(base + c * _C, _C)], sos[b])
            @pl.when(c + _NB < _CHUNKS)
            def _():
                start_in(c + _NB, b)

    for b in range(_NB):
        pltpu.make_async_copy(obufs[b], o_hbm.at[pl.ds(base, _C)],
                              sos[b]).wait()


@jax.jit
def kernel(x, target):
    mesh = plsc.VectorSubcoreMesh(core_axis_name="c", subcore_axis_name="s")
    run = functools.partial(
        pl.kernel,
        mesh=mesh,
        out_type=jax.ShapeDtypeStruct((_N,), jnp.int32),
        scratch_types=[
            [pltpu.VMEM((_C,), jnp.float32) for _ in range(_NB)],
            [pltpu.VMEM((_C,), jnp.float32) for _ in range(_NB)],
            [pltpu.VMEM((_C,), jnp.int32) for _ in range(_NB)],
            [pltpu.SemaphoreType.DMA for _ in range(_NB)],
            [pltpu.SemaphoreType.DMA for _ in range(_NB)],
            [pltpu.SemaphoreType.DMA for _ in range(_NB)],
        ],
    )(_sc_body)
    return run(x, target)
